# Initial kernel scaffold; baseline (speedup 1.0000x reference)
import jax
import jax.numpy as jnp
from jax.experimental import pallas as pl


def kernel(x, edge_index, edge_attr, edge_index2, edge_attr2, batch, num_graphs, W_ln, W_c1, W_c2, W_qkv, b_qkv, W_o, b_o, ln_g, ln_b):
    raise NotImplementedError("write your pallas kernel here")


# speedup vs baseline: 11.9465x; 11.9465x over previous
import functools

import jax
import jax.numpy as jnp
from jax import lax
from jax.experimental import pallas as pl
from jax.experimental.pallas import tpu as pltpu
from jax.experimental.pallas import tpu_sc as plsc

N = 10000
D = 128
NHEADS = 4
HD = D // NHEADS
NG = 8

SC_CORES = 2
SC_SUBCORES = 16
LANES = 16

E_SET = 320000
E_PAD = 327680
E_PER_SUB = E_PAD // SC_SUBCORES
CHUNK = 128
N_CHUNKS = E_PER_SUB // CHUNK
N_FULL_OUT = N // CHUNK
N_TAIL_OUT = N - N_FULL_OUT * CHUNK
N_OUT_TURNS = N_FULL_OUT // SC_SUBCORES + 1

NPAD = 10240
BR = 400
BC = 512
NRB = N // BR
NCB = NPAD // BC
MM_BR = 640


def _sc_scatter_build():
    mesh = plsc.VectorSubcoreMesh(core_axis_name="c", subcore_axis_name="s",
                                  num_cores=SC_CORES, num_subcores=SC_SUBCORES)

    @functools.partial(
        pl.kernel,
        mesh=mesh,
        out_type=jax.ShapeDtypeStruct((SC_CORES, N, D), jnp.float32),
        scratch_types=[
            pltpu.VMEM((CHUNK, D), jnp.float32),
            pltpu.VMEM((CHUNK,), jnp.int32),
            pltpu.VMEM((CHUNK,), jnp.int32),
            pltpu.VMEM((CHUNK,), jnp.float32),
            pltpu.VMEM_SHARED((N, D), jnp.float32),
            pltpu.SemaphoreType.DMA,
        ],
    )
    def sc_kernel(x_hbm, src_hbm, dst_hbm, ea_hbm, out_hbm,
                  rows_v, src_v, dst_v, ea_v, acc_sh, sem):
        cc = lax.axis_index("c")
        ss = lax.axis_index("s")

        def zero_body(i, carry):
            for d in range(D // LANES):
                rows_v[i, pl.ds(d * LANES, LANES)] = jnp.zeros((LANES,), jnp.float32)
            return carry
        lax.fori_loop(0, CHUNK, zero_body, 0)
        for t in range(N_OUT_TURNS):
            idx = ss + SC_SUBCORES * t

            @pl.when(idx < N_FULL_OUT)
            def _():
                pltpu.sync_copy(rows_v, acc_sh.at[pl.ds(idx * CHUNK, CHUNK)])

            @pl.when(idx == N_FULL_OUT)
            def _():
                pltpu.sync_copy(rows_v.at[pl.ds(0, N_TAIL_OUT)],
                                acc_sh.at[pl.ds(N_FULL_OUT * CHUNK, N_TAIL_OUT)])
        plsc.subcore_barrier()

        base0 = cc * E_PAD + ss * E_PER_SUB

        def chunk_body(k, carry):
            b = base0 + k * CHUNK
            pltpu.sync_copy(src_hbm.at[pl.ds(b, CHUNK)], src_v)
            pltpu.sync_copy(dst_hbm.at[pl.ds(b, CHUNK)], dst_v)
            pltpu.sync_copy(ea_hbm.at[pl.ds(b, CHUNK)], ea_v)
            pltpu.async_copy(x_hbm.at[src_v], rows_v, sem).wait()

            def scale_body(g, c2):
                ev = ea_v[pl.ds(g * LANES, LANES)]
                for i in range(LANES):
                    eav = ev[i] * jnp.ones((LANES,), jnp.float32)
                    e = g * LANES + i
                    for d in range(D // LANES):
                        sl = pl.ds(d * LANES, LANES)
                        rows_v[e, sl] = rows_v[e, sl] * eav
                return c2
            lax.fori_loop(0, CHUNK // LANES, scale_body, 0)

            pltpu.sync_copy(rows_v, acc_sh.at[dst_v], add=True)
            return carry
        lax.fori_loop(0, N_CHUNKS, chunk_body, 0)
        plsc.subcore_barrier()

        for t in range(N_OUT_TURNS):
            idx = ss + SC_SUBCORES * t

            @pl.when(idx < N_FULL_OUT)
            def _():
                off = idx * CHUNK
                pltpu.sync_copy(acc_sh.at[pl.ds(off, CHUNK)], rows_v)
                pltpu.sync_copy(rows_v, out_hbm.at[cc, pl.ds(off, CHUNK)])

            @pl.when(idx == N_FULL_OUT)
            def _():
                off = N_FULL_OUT * CHUNK
                pltpu.sync_copy(acc_sh.at[pl.ds(off, N_TAIL_OUT)],
                                rows_v.at[pl.ds(0, N_TAIL_OUT)])
                pltpu.sync_copy(rows_v.at[pl.ds(0, N_TAIL_OUT)],
                                out_hbm.at[cc, pl.ds(off, N_TAIL_OUT)])

    return sc_kernel


def _wfold_body(wcat_ref, wqkv_ref, o_ref):
    o_ref[:, :D] = wcat_ref[...]
    o_ref[:, D:] = jnp.dot(wcat_ref[...], wqkv_ref[...],
                           preferred_element_type=jnp.float32)


def _wfold(wcat, wqkv):
    return pl.pallas_call(
        _wfold_body,
        out_shape=jax.ShapeDtypeStruct((3 * D, 4 * D), jnp.float32),
    )(wcat, wqkv)


def _mm_body(a_ref, w_ref, b_ref, o_ref):
    o_ref[...] = jnp.dot(a_ref[...], w_ref[...],
                         preferred_element_type=jnp.float32) + b_ref[...]


def _mm(cat_pad, w_big, bias):
    return pl.pallas_call(
        _mm_body,
        grid=(NPAD // MM_BR,),
        in_specs=[
            pl.BlockSpec((MM_BR, 3 * D), lambda i: (i, 0)),
            pl.BlockSpec((3 * D, 4 * D), lambda i: (0, 0)),
            pl.BlockSpec((1, 4 * D), lambda i: (0, 0)),
        ],
        out_specs=pl.BlockSpec((MM_BR, 4 * D), lambda i: (i, 0)),
        out_shape=jax.ShapeDtypeStruct((NPAD, 4 * D), jnp.float32),
    )(cat_pad, w_big, bias)


def _attn_body(lo_ref, hi_ref, iq_ref, brow_ref, bcol_ref,
               wo_ref, bo_ref, lng_ref, lnb_ref, o_ref):
    r = pl.program_id(0)
    rbase = pl.multiple_of(r * BR, BR)
    brow = brow_ref[...]
    scale = 1.0 / jnp.sqrt(jnp.float32(HD))

    qs = [iq_ref[pl.ds(rbase, BR), D + h * HD: D + (h + 1) * HD]
          for h in range(NHEADS)]

    m0 = jnp.full((BR, 1), -1e30, jnp.float32)
    l0 = jnp.zeros((BR, 1), jnp.float32)
    a0 = jnp.zeros((BR, HD), jnp.float32)
    carry0 = tuple([m0] * NHEADS + [l0] * NHEADS + [a0] * NHEADS)

    def col_step(j, carry):
        ms = list(carry[:NHEADS])
        ls = list(carry[NHEADS:2 * NHEADS])
        accs = list(carry[2 * NHEADS:])
        cbase = pl.multiple_of(j * BC, BC)
        bcol = bcol_ref[pl.ds(j, 1), :]
        mask = brow == bcol
        for h in range(NHEADS):
            kh = iq_ref[pl.ds(cbase, BC), 2 * D + h * HD: 2 * D + (h + 1) * HD]
            vh = iq_ref[pl.ds(cbase, BC), 3 * D + h * HD: 3 * D + (h + 1) * HD]
            s = lax.dot_general(qs[h], kh, (((1,), (1,)), ((), ())),
                                preferred_element_type=jnp.float32) * scale
            s = jnp.where(mask, s, jnp.float32(-1e30))
            m_new = jnp.maximum(ms[h], jnp.max(s, axis=1, keepdims=True))
            alpha = jnp.exp(ms[h] - m_new)
            p = jnp.exp(s - m_new)
            p = jnp.where(mask, p, jnp.float32(0.0))
            ls[h] = ls[h] * alpha + jnp.sum(p, axis=1, keepdims=True)
            accs[h] = accs[h] * alpha + lax.dot_general(
                p, vh, (((1,), (0,)), ((), ())),
                preferred_element_type=jnp.float32)
            ms[h] = m_new
        return tuple(ms + ls + accs)

    carry = lax.fori_loop(lo_ref[r], hi_ref[r], col_step, carry0)
    ls = carry[NHEADS:2 * NHEADS]
    accs = carry[2 * NHEADS:]
    att = jnp.concatenate([accs[h] / ls[h] for h in range(NHEADS)], axis=1)

    o = jnp.dot(att, wo_ref[...], preferred_element_type=jnp.float32) + bo_ref[...]
    hres = iq_ref[pl.ds(rbase, BR), :D] + o
    mu = jnp.mean(hres, axis=1, keepdims=True)
    dlt = hres - mu
    var = jnp.mean(dlt * dlt, axis=1, keepdims=True)
    o_ref[...] = dlt * lax.rsqrt(var + 1e-5) * lng_ref[...] + lnb_ref[...]


def _attn(lo_blk, hi_blk, incqkv_pad, brow2d, bcol2d, w_o, bo2d, lng2d, lnb2d):
    return pl.pallas_call(
        _attn_body,
        grid=(NRB,),
        in_specs=[
            pl.BlockSpec(memory_space=pltpu.SMEM),
            pl.BlockSpec(memory_space=pltpu.SMEM),
            pl.BlockSpec((NPAD, 4 * D), lambda r: (0, 0)),
            pl.BlockSpec((BR, 1), lambda r: (r, 0)),
            pl.BlockSpec((NCB, BC), lambda r: (0, 0)),
            pl.BlockSpec((D, D), lambda r: (0, 0)),
            pl.BlockSpec((1, D), lambda r: (0, 0)),
            pl.BlockSpec((1, D), lambda r: (0, 0)),
            pl.BlockSpec((1, D), lambda r: (0, 0)),
        ],
        out_specs=pl.BlockSpec((BR, D), lambda r: (r, 0)),
        out_shape=jax.ShapeDtypeStruct((N, D), jnp.float32),
    )(lo_blk, hi_blk, incqkv_pad, brow2d, bcol2d, w_o, bo2d, lng2d, lnb2d)


def kernel(x, edge_index, edge_attr, edge_index2, edge_attr2, batch, num_graphs,
           W_ln, W_c1, W_c2, W_qkv, b_qkv, W_o, b_o, ln_g, ln_b):
    ep = E_PAD - E_SET
    src_all = jnp.concatenate([jnp.pad(edge_index[0], (0, ep)),
                               jnp.pad(edge_index2[0], (0, ep))])
    dst_all = jnp.concatenate([jnp.pad(edge_index[1], (0, ep)),
                               jnp.pad(edge_index2[1], (0, ep))])
    ea_all = jnp.concatenate([jnp.pad(edge_attr, (0, ep)),
                              jnp.pad(edge_attr2, (0, ep))])

    y_both = _sc_scatter_build()(x, src_all, dst_all, ea_all)

    wcat = jnp.concatenate([W_ln, W_c1, W_c2], axis=0)
    w_big = _wfold(wcat, W_qkv)
    bias = jnp.concatenate([jnp.zeros((D,), jnp.float32), b_qkv])[None, :]

    cat = jnp.concatenate([x, y_both[0], y_both[1]], axis=1)
    cat_pad = jnp.pad(cat, ((0, NPAD - N), (0, 0)))
    incqkv_pad = _mm(cat_pad, w_big, bias)

    gids = jnp.arange(NG, dtype=batch.dtype)
    counts = jnp.sum(batch[None, :] == gids[:, None], axis=1).astype(jnp.int32)
    starts = jnp.cumsum(counts) - counts
    rb = batch.reshape(NRB, BR)
    bfirst = rb[:, 0]
    blast = rb[:, -1]
    col_lo = jnp.take(starts, bfirst)
    col_hi = jnp.take(starts, blast) + jnp.take(counts, blast)
    lo_blk = (col_lo // BC).astype(jnp.int32)
    hi_blk = ((col_hi + BC - 1) // BC).astype(jnp.int32)

    brow2d = batch[:, None].astype(jnp.int32)
    bcol2d = jnp.pad(batch.astype(jnp.int32), (0, NPAD - N),
                     constant_values=-1).reshape(NCB, BC)

    out = _attn(lo_blk, hi_blk, incqkv_pad, brow2d, bcol2d,
                W_o, b_o[None, :], ln_g[None, :], ln_b[None, :])
    return out


# speedup vs baseline: 21.8021x; 1.8250x over previous
import functools

import jax
import jax.numpy as jnp
from jax import lax
from jax.experimental import pallas as pl
from jax.experimental.pallas import tpu as pltpu
from jax.experimental.pallas import tpu_sc as plsc

N = 10000
D = 128
NHEADS = 4
HD = D // NHEADS
NG = 8

SC_CORES = 2
SC_SUBCORES = 16
LANES = 16

E_SET = 320000
CHUNK = 112
N_CHUNKS = 180
E_PER_SUB = N_CHUNKS * CHUNK
E_PAD = E_PER_SUB * SC_SUBCORES
CHUNKS_PER_SET = E_PAD // CHUNK
N_FULL_OUT = N // CHUNK
N_TAIL_OUT = N - N_FULL_OUT * CHUNK
N_OUT_TURNS = N_FULL_OUT // SC_SUBCORES + 1

NPAD = 10240
BR = 400
BC = 512
NRB = N // BR
NCB = NPAD // BC
MM_BR = 640


def _sc_scatter_build():
    mesh = plsc.VectorSubcoreMesh(core_axis_name="c", subcore_axis_name="s",
                                  num_cores=SC_CORES, num_subcores=SC_SUBCORES)

    @functools.partial(
        pl.kernel,
        mesh=mesh,
        out_type=jax.ShapeDtypeStruct((SC_CORES, N, D), jnp.float32),
        scratch_types=[
            pltpu.VMEM((CHUNK, D), jnp.float32),
            pltpu.VMEM((CHUNK, D), jnp.float32),
            pltpu.VMEM((CHUNK, D), jnp.float32),
            pltpu.VMEM((3, CHUNK), jnp.int32),
            pltpu.VMEM((3, CHUNK), jnp.float32),
            pltpu.VMEM((6, CHUNK), jnp.int32),
            pltpu.VMEM_SHARED((N, D), jnp.float32),
        ] + [pltpu.SemaphoreType.DMA] * 18,
    )
    def sc_kernel(x_hbm, src_hbm, dst_hbm, ea_hbm, out_hbm,
                  r0, r1, r2, src_r, ea_r, dst_r, acc_sh, *sems):
        cc = lax.axis_index("c")
        ss = lax.axis_index("s")
        rows = [r0, r1, r2]
        gsem = list(sems[0:3])
        ssem = list(sems[3:6])
        msem = list(sems[6:9])
        esem = list(sems[9:12])
        dsem = list(sems[12:18])
        rows_v = r0

        def zero_body(i, carry):
            for d in range(D // LANES):
                rows_v[i, pl.ds(d * LANES, LANES)] = jnp.zeros((LANES,), jnp.float32)
            return carry
        lax.fori_loop(0, CHUNK, zero_body, 0)
        for t in range(N_OUT_TURNS):
            idx = ss + SC_SUBCORES * t

            @pl.when(idx < N_FULL_OUT)
            def _():
                pltpu.sync_copy(rows_v, acc_sh.at[pl.ds(idx * CHUNK, CHUNK)])

            @pl.when(idx == N_FULL_OUT)
            def _():
                pltpu.sync_copy(rows_v.at[pl.ds(0, N_TAIL_OUT)],
                                acc_sh.at[pl.ds(N_FULL_OUT * CHUNK, N_TAIL_OUT)])
        plsc.subcore_barrier()

        ebase = (cc * SC_SUBCORES + ss) * E_PER_SUB

        def start_idx(c, j3, j6):
            off = ebase + c * CHUNK
            pltpu.async_copy(src_hbm.at[pl.ds(off, CHUNK)], src_r.at[j3], msem[j3])
            pltpu.async_copy(ea_hbm.at[pl.ds(off, CHUNK)], ea_r.at[j3], esem[j3])
            pltpu.async_copy(dst_hbm.at[pl.ds(off, CHUNK)], dst_r.at[j6], dsem[j6])

        def wait_src(j3):
            pltpu.make_async_copy(src_hbm.at[pl.ds(0, CHUNK)], src_r.at[j3],
                                  msem[j3]).wait()

        def wait_ea(j3):
            pltpu.make_async_copy(ea_hbm.at[pl.ds(0, CHUNK)], ea_r.at[j3],
                                  esem[j3]).wait()

        def wait_dst(j6):
            pltpu.make_async_copy(dst_hbm.at[pl.ds(0, CHUNK)], dst_r.at[j6],
                                  dsem[j6]).wait()

        def start_gather(j3):
            pltpu.async_copy(x_hbm.at[src_r.at[j3]], rows[j3], gsem[j3])

        def wait_gather(j3):
            pltpu.make_async_copy(x_hbm.at[pl.ds(0, CHUNK)], rows[j3],
                                  gsem[j3]).wait()

        def start_scatter(j6, j3):
            pltpu.async_copy(rows[j3], acc_sh.at[dst_r.at[j6]], ssem[j3],
                             add=True)

        def wait_scatter(j3):
            pltpu.make_async_copy(x_hbm.at[pl.ds(0, CHUNK)], rows[j3],
                                  ssem[j3]).wait()

        def scale_chunk(j3):
            buf = rows[j3]

            def scale_body(g, c2):
                ev = ea_r[j3, pl.ds(g * LANES, LANES)]
                for i in range(LANES):
                    eav = ev[i] * jnp.ones((LANES,), jnp.float32)
                    e = g * LANES + i
                    for d in range(D // LANES):
                        sl = pl.ds(d * LANES, LANES)
                        buf[e, sl] = buf[e, sl] * eav
                return c2
            lax.fori_loop(0, CHUNK // LANES, scale_body, 0)

        start_idx(0, 0, 0)
        start_idx(1, 1, 1)
        start_idx(2, 2, 2)
        wait_src(0)
        start_gather(0)
        wait_src(1)
        start_gather(1)
        n_outer = N_CHUNKS // 6

        def outer_body(t, carry):
            for j in range(6):
                c = 6 * t + j
                j3 = j % 3
                jp = (j + 2) % 3
                wait_gather(j3)
                wait_ea(j3)
                scale_chunk(j3)
                wait_dst(j)
                start_scatter(j, j3)

                def prefetch():
                    wait_src(jp)
                    wait_scatter(jp)
                    start_gather(jp)

                def refill():
                    start_idx(c + 3, j3, (j + 3) % 6)

                if j == 0:
                    wait_src(jp)

                    @pl.when(t > 0)
                    def _():
                        wait_scatter(jp)
                    start_gather(jp)
                    refill()
                elif j <= 2:
                    prefetch()
                    refill()
                elif j == 3:
                    prefetch()

                    @pl.when(t < n_outer - 1)
                    def _():
                        refill()
                else:
                    @pl.when(t < n_outer - 1)
                    def _():
                        prefetch()
                        refill()
            return carry
        lax.fori_loop(0, n_outer, outer_body, 0)
        for j3 in range(3):
            wait_scatter(j3)
        plsc.subcore_barrier()

        for t in range(N_OUT_TURNS):
            idx = ss + SC_SUBCORES * t

            @pl.when(idx < N_FULL_OUT)
            def _():
                off = idx * CHUNK
                pltpu.sync_copy(acc_sh.at[pl.ds(off, CHUNK)], rows_v)
                pltpu.sync_copy(rows_v, out_hbm.at[cc, pl.ds(off, CHUNK)])

            @pl.when(idx == N_FULL_OUT)
            def _():
                off = N_FULL_OUT * CHUNK
                pltpu.sync_copy(acc_sh.at[pl.ds(off, N_TAIL_OUT)],
                                rows_v.at[pl.ds(0, N_TAIL_OUT)])
                pltpu.sync_copy(rows_v.at[pl.ds(0, N_TAIL_OUT)],
                                out_hbm.at[cc, pl.ds(off, N_TAIL_OUT)])

    return sc_kernel


def _wfold_body(wcat_ref, wqkv_ref, o_ref):
    o_ref[:, :D] = wcat_ref[...]
    o_ref[:, D:] = jnp.dot(wcat_ref[...], wqkv_ref[...],
                           preferred_element_type=jnp.float32)


def _wfold(wcat, wqkv):
    return pl.pallas_call(
        _wfold_body,
        out_shape=jax.ShapeDtypeStruct((3 * D, 4 * D), jnp.float32),
    )(wcat, wqkv)


def _mm_body(a_ref, w_ref, b_ref, o_ref):
    o_ref[...] = jnp.dot(a_ref[...], w_ref[...],
                         preferred_element_type=jnp.float32) + b_ref[...]


def _mm(cat_pad, w_big, bias):
    return pl.pallas_call(
        _mm_body,
        grid=(NPAD // MM_BR,),
        in_specs=[
            pl.BlockSpec((MM_BR, 3 * D), lambda i: (i, 0)),
            pl.BlockSpec((3 * D, 4 * D), lambda i: (0, 0)),
            pl.BlockSpec((1, 4 * D), lambda i: (0, 0)),
        ],
        out_specs=pl.BlockSpec((MM_BR, 4 * D), lambda i: (i, 0)),
        out_shape=jax.ShapeDtypeStruct((NPAD, 4 * D), jnp.float32),
    )(cat_pad, w_big, bias)


def _attn_body(lo_ref, hi_ref, iq_ref, brow_ref, bcol_ref,
               wo_ref, bo_ref, lng_ref, lnb_ref, o_ref):
    r = pl.program_id(0)
    rbase = pl.multiple_of(r * BR, BR)
    brow = brow_ref[...]
    scale = 1.0 / jnp.sqrt(jnp.float32(HD))

    qs = [iq_ref[pl.ds(rbase, BR), D + h * HD: D + (h + 1) * HD]
          for h in range(NHEADS)]

    m0 = jnp.full((BR, 1), -1e30, jnp.float32)
    l0 = jnp.zeros((BR, 1), jnp.float32)
    a0 = jnp.zeros((BR, HD), jnp.float32)
    carry0 = tuple([m0] * NHEADS + [l0] * NHEADS + [a0] * NHEADS)

    def col_step(j, carry):
        ms = list(carry[:NHEADS])
        ls = list(carry[NHEADS:2 * NHEADS])
        accs = list(carry[2 * NHEADS:])
        cbase = pl.multiple_of(j * BC, BC)
        bcol = bcol_ref[pl.ds(j, 1), :]
        mask = brow == bcol
        for h in range(NHEADS):
            kh = iq_ref[pl.ds(cbase, BC), 2 * D + h * HD: 2 * D + (h + 1) * HD]
            vh = iq_ref[pl.ds(cbase, BC), 3 * D + h * HD: 3 * D + (h + 1) * HD]
            s = lax.dot_general(qs[h], kh, (((1,), (1,)), ((), ())),
                                preferred_element_type=jnp.float32) * scale
            s = jnp.where(mask, s, jnp.float32(-1e30))
            m_new = jnp.maximum(ms[h], jnp.max(s, axis=1, keepdims=True))
            alpha = jnp.exp(ms[h] - m_new)
            p = jnp.exp(s - m_new)
            p = jnp.where(mask, p, jnp.float32(0.0))
            ls[h] = ls[h] * alpha + jnp.sum(p, axis=1, keepdims=True)
            accs[h] = accs[h] * alpha + lax.dot_general(
                p, vh, (((1,), (0,)), ((), ())),
                preferred_element_type=jnp.float32)
            ms[h] = m_new
        return tuple(ms + ls + accs)

    carry = lax.fori_loop(lo_ref[r], hi_ref[r], col_step, carry0)
    ls = carry[NHEADS:2 * NHEADS]
    accs = carry[2 * NHEADS:]
    att = jnp.concatenate([accs[h] / ls[h] for h in range(NHEADS)], axis=1)

    o = jnp.dot(att, wo_ref[...], preferred_element_type=jnp.float32) + bo_ref[...]
    hres = iq_ref[pl.ds(rbase, BR), :D] + o
    mu = jnp.mean(hres, axis=1, keepdims=True)
    dlt = hres - mu
    var = jnp.mean(dlt * dlt, axis=1, keepdims=True)
    o_ref[...] = dlt * lax.rsqrt(var + 1e-5) * lng_ref[...] + lnb_ref[...]


def _attn(lo_blk, hi_blk, incqkv_pad, brow2d, bcol2d, w_o, bo2d, lng2d, lnb2d):
    return pl.pallas_call(
        _attn_body,
        grid=(NRB,),
        in_specs=[
            pl.BlockSpec(memory_space=pltpu.SMEM),
            pl.BlockSpec(memory_space=pltpu.SMEM),
            pl.BlockSpec((NPAD, 4 * D), lambda r: (0, 0)),
            pl.BlockSpec((BR, 1), lambda r: (r, 0)),
            pl.BlockSpec((NCB, BC), lambda r: (0, 0)),
            pl.BlockSpec((D, D), lambda r: (0, 0)),
            pl.BlockSpec((1, D), lambda r: (0, 0)),
            pl.BlockSpec((1, D), lambda r: (0, 0)),
            pl.BlockSpec((1, D), lambda r: (0, 0)),
        ],
        out_specs=pl.BlockSpec((BR, D), lambda r: (r, 0)),
        out_shape=jax.ShapeDtypeStruct((N, D), jnp.float32),
    )(lo_blk, hi_blk, incqkv_pad, brow2d, bcol2d, w_o, bo2d, lng2d, lnb2d)


def kernel(x, edge_index, edge_attr, edge_index2, edge_attr2, batch, num_graphs,
           W_ln, W_c1, W_c2, W_qkv, b_qkv, W_o, b_o, ln_g, ln_b):
    ep = E_PAD - E_SET
    src_all = jnp.concatenate([jnp.pad(edge_index[0], (0, ep)),
                               jnp.pad(edge_index2[0], (0, ep))])
    dst_all = jnp.concatenate([jnp.pad(edge_index[1], (0, ep)),
                               jnp.pad(edge_index2[1], (0, ep))])
    ea_all = jnp.concatenate([jnp.pad(edge_attr, (0, ep)),
                              jnp.pad(edge_attr2, (0, ep))])

    y_both = _sc_scatter_build()(x, src_all, dst_all, ea_all)

    wcat = jnp.concatenate([W_ln, W_c1, W_c2], axis=0)
    w_big = _wfold(wcat, W_qkv)
    bias = jnp.concatenate([jnp.zeros((D,), jnp.float32), b_qkv])[None, :]

    cat = jnp.concatenate([x, y_both[0], y_both[1]], axis=1)
    cat_pad = jnp.pad(cat, ((0, NPAD - N), (0, 0)))
    incqkv_pad = _mm(cat_pad, w_big, bias)

    gids = jnp.arange(NG, dtype=batch.dtype)
    counts = jnp.sum(batch[None, :] == gids[:, None], axis=1).astype(jnp.int32)
    starts = jnp.cumsum(counts) - counts
    rb = batch.reshape(NRB, BR)
    bfirst = rb[:, 0]
    blast = rb[:, -1]
    col_lo = jnp.take(starts, bfirst)
    col_hi = jnp.take(starts, blast) + jnp.take(counts, blast)
    lo_blk = (col_lo // BC).astype(jnp.int32)
    hi_blk = ((col_hi + BC - 1) // BC).astype(jnp.int32)

    brow2d = batch[:, None].astype(jnp.int32)
    bcol2d = jnp.pad(batch.astype(jnp.int32), (0, NPAD - N),
                     constant_values=-1).reshape(NCB, BC)

    out = _attn(lo_blk, hi_blk, incqkv_pad, brow2d, bcol2d,
                W_o, b_o[None, :], ln_g[None, :], ln_b[None, :])
    return out


# speedup vs baseline: 23.5049x; 1.0781x over previous
import functools

import jax
import jax.numpy as jnp
from jax import lax
from jax.experimental import pallas as pl
from jax.experimental.pallas import tpu as pltpu
from jax.experimental.pallas import tpu_sc as plsc

N = 10000
D = 128
NHEADS = 4
HD = D // NHEADS
NG = 8

SC_CORES = 2
SC_SUBCORES = 16
LANES = 16

E_SET = 320000
CHUNK = 112
N_CHUNKS = 180
E_PER_SUB = N_CHUNKS * CHUNK
E_PAD = E_PER_SUB * SC_SUBCORES
CHUNKS_PER_SET = E_PAD // CHUNK
N_FULL_OUT = N // CHUNK
N_TAIL_OUT = N - N_FULL_OUT * CHUNK
N_OUT_TURNS = N_FULL_OUT // SC_SUBCORES + 1

BR = 400
BC = 400
NRB = N // BR
NCB = N // BC
MM_BR = 2000


def _sc_scatter_build():
    mesh = plsc.VectorSubcoreMesh(core_axis_name="c", subcore_axis_name="s",
                                  num_cores=SC_CORES, num_subcores=SC_SUBCORES)

    @functools.partial(
        pl.kernel,
        mesh=mesh,
        out_type=jax.ShapeDtypeStruct((SC_CORES, N, D), jnp.float32),
        scratch_types=[
            pltpu.VMEM((CHUNK, D), jnp.float32),
            pltpu.VMEM((CHUNK, D), jnp.float32),
            pltpu.VMEM((CHUNK, D), jnp.float32),
            pltpu.VMEM((3, CHUNK), jnp.int32),
            pltpu.VMEM((3, CHUNK), jnp.float32),
            pltpu.VMEM((6, CHUNK), jnp.int32),
            pltpu.VMEM_SHARED((N, D), jnp.float32),
        ] + [pltpu.SemaphoreType.DMA] * 18,
    )
    def sc_kernel(x_hbm, src_hbm, dst_hbm, ea_hbm, out_hbm,
                  r0, r1, r2, src_r, ea_r, dst_r, acc_sh, *sems):
        cc = lax.axis_index("c")
        ss = lax.axis_index("s")
        rows = [r0, r1, r2]
        gsem = list(sems[0:3])
        ssem = list(sems[3:6])
        msem = list(sems[6:9])
        esem = list(sems[9:12])
        dsem = list(sems[12:18])
        rows_v = r0

        def zero_body(i, carry):
            for d in range(D // LANES):
                rows_v[i, pl.ds(d * LANES, LANES)] = jnp.zeros((LANES,), jnp.float32)
            return carry
        lax.fori_loop(0, CHUNK, zero_body, 0)
        for t in range(N_OUT_TURNS):
            idx = ss + SC_SUBCORES * t

            @pl.when(idx < N_FULL_OUT)
            def _():
                pltpu.sync_copy(rows_v, acc_sh.at[pl.ds(idx * CHUNK, CHUNK)])

            @pl.when(idx == N_FULL_OUT)
            def _():
                pltpu.sync_copy(rows_v.at[pl.ds(0, N_TAIL_OUT)],
                                acc_sh.at[pl.ds(N_FULL_OUT * CHUNK, N_TAIL_OUT)])
        plsc.subcore_barrier()

        ebase = (cc * SC_SUBCORES + ss) * E_PER_SUB

        def start_idx(c, j3, j6):
            off = ebase + c * CHUNK
            pltpu.async_copy(src_hbm.at[pl.ds(off, CHUNK)], src_r.at[j3], msem[j3])
            pltpu.async_copy(ea_hbm.at[pl.ds(off, CHUNK)], ea_r.at[j3], esem[j3])
            pltpu.async_copy(dst_hbm.at[pl.ds(off, CHUNK)], dst_r.at[j6], dsem[j6])

        def wait_src(j3):
            pltpu.make_async_copy(src_hbm.at[pl.ds(0, CHUNK)], src_r.at[j3],
                                  msem[j3]).wait()

        def wait_ea(j3):
            pltpu.make_async_copy(ea_hbm.at[pl.ds(0, CHUNK)], ea_r.at[j3],
                                  esem[j3]).wait()

        def wait_dst(j6):
            pltpu.make_async_copy(dst_hbm.at[pl.ds(0, CHUNK)], dst_r.at[j6],
                                  dsem[j6]).wait()

        def start_gather(j3):
            pltpu.async_copy(x_hbm.at[src_r.at[j3]], rows[j3], gsem[j3])

        def wait_gather(j3):
            pltpu.make_async_copy(x_hbm.at[pl.ds(0, CHUNK)], rows[j3],
                                  gsem[j3]).wait()

        def start_scatter(j6, j3):
            pltpu.async_copy(rows[j3], acc_sh.at[dst_r.at[j6]], ssem[j3],
                             add=True)

        def wait_scatter(j3):
            pltpu.make_async_copy(x_hbm.at[pl.ds(0, CHUNK)], rows[j3],
                                  ssem[j3]).wait()

        def scale_chunk(j3):
            buf = rows[j3]

            def scale_body(g, c2):
                ev = ea_r[j3, pl.ds(g * LANES, LANES)]
                for i in range(LANES):
                    eav = ev[i] * jnp.ones((LANES,), jnp.float32)
                    e = g * LANES + i
                    for d in range(D // LANES):
                        sl = pl.ds(d * LANES, LANES)
                        buf[e, sl] = buf[e, sl] * eav
                return c2
            lax.fori_loop(0, CHUNK // LANES, scale_body, 0)

        start_idx(0, 0, 0)
        start_idx(1, 1, 1)
        start_idx(2, 2, 2)
        wait_src(0)
        start_gather(0)
        wait_src(1)
        start_gather(1)
        n_outer = N_CHUNKS // 6

        def outer_body(t, carry):
            for j in range(6):
                c = 6 * t + j
                j3 = j % 3
                jp = (j + 2) % 3
                wait_gather(j3)
                wait_ea(j3)
                scale_chunk(j3)
                wait_dst(j)
                start_scatter(j, j3)

                def prefetch():
                    wait_src(jp)
                    wait_scatter(jp)
                    start_gather(jp)

                def refill():
                    start_idx(c + 3, j3, (j + 3) % 6)

                if j == 0:
                    wait_src(jp)

                    @pl.when(t > 0)
                    def _():
                        wait_scatter(jp)
                    start_gather(jp)
                    refill()
                elif j <= 2:
                    prefetch()
                    refill()
                elif j == 3:
                    prefetch()

                    @pl.when(t < n_outer - 1)
                    def _():
                        refill()
                else:
                    @pl.when(t < n_outer - 1)
                    def _():
                        prefetch()
                        refill()
            return carry
        lax.fori_loop(0, n_outer, outer_body, 0)
        for j3 in range(3):
            wait_scatter(j3)
        plsc.subcore_barrier()

        for t in range(N_OUT_TURNS):
            idx = ss + SC_SUBCORES * t

            @pl.when(idx < N_FULL_OUT)
            def _():
                off = idx * CHUNK
                pltpu.sync_copy(acc_sh.at[pl.ds(off, CHUNK)], rows_v)
                pltpu.sync_copy(rows_v, out_hbm.at[cc, pl.ds(off, CHUNK)])

            @pl.when(idx == N_FULL_OUT)
            def _():
                off = N_FULL_OUT * CHUNK
                pltpu.sync_copy(acc_sh.at[pl.ds(off, N_TAIL_OUT)],
                                rows_v.at[pl.ds(0, N_TAIL_OUT)])
                pltpu.sync_copy(rows_v.at[pl.ds(0, N_TAIL_OUT)],
                                out_hbm.at[cc, pl.ds(off, N_TAIL_OUT)])

    return sc_kernel


def _wfold_body(wcat_ref, wqkv_ref, o_ref):
    o_ref[:, :D] = wcat_ref[...]
    o_ref[:, D:] = jnp.dot(wcat_ref[...], wqkv_ref[...],
                           preferred_element_type=jnp.float32)


def _wfold(wcat, wqkv):
    return pl.pallas_call(
        _wfold_body,
        out_shape=jax.ShapeDtypeStruct((3 * D, 4 * D), jnp.float32),
    )(wcat, wqkv)


def _mm_body(x_ref, y_ref, w_ref, b_ref, o_ref):
    acc = jnp.dot(x_ref[...], w_ref[:D, :], preferred_element_type=jnp.float32)
    acc += jnp.dot(y_ref[0], w_ref[D:2 * D, :], preferred_element_type=jnp.float32)
    acc += jnp.dot(y_ref[1], w_ref[2 * D:, :], preferred_element_type=jnp.float32)
    o_ref[...] = acc + b_ref[...]


def _mm(x, y_both, w_big, bias):
    return pl.pallas_call(
        _mm_body,
        grid=(N // MM_BR,),
        in_specs=[
            pl.BlockSpec((MM_BR, D), lambda i: (i, 0)),
            pl.BlockSpec((2, MM_BR, D), lambda i: (0, i, 0)),
            pl.BlockSpec((3 * D, 4 * D), lambda i: (0, 0)),
            pl.BlockSpec((1, 4 * D), lambda i: (0, 0)),
        ],
        out_specs=pl.BlockSpec((MM_BR, 4 * D), lambda i: (i, 0)),
        out_shape=jax.ShapeDtypeStruct((N, 4 * D), jnp.float32),
    )(x, y_both, w_big, bias)


def _attn_body(lo_ref, hi_ref, iq_ref, brow_ref, bcol_ref,
               wo_ref, bo_ref, lng_ref, lnb_ref, o_ref):
    r = pl.program_id(0)
    rbase = pl.multiple_of(r * BR, BR)
    brow = brow_ref[...]
    scale = 1.0 / jnp.sqrt(jnp.float32(HD))

    qs = [iq_ref[pl.ds(rbase, BR), D + h * HD: D + (h + 1) * HD] * scale
          for h in range(NHEADS)]
    ones_col = jnp.ones((BC, 1), jnp.float32)

    l0 = jnp.zeros((BR, 1), jnp.float32)
    a0 = jnp.zeros((BR, HD), jnp.float32)
    carry0 = tuple([l0] * NHEADS + [a0] * NHEADS)

    def col_step(j, carry):
        ls = list(carry[:NHEADS])
        accs = list(carry[NHEADS:])
        cbase = pl.multiple_of(j * BC, BC)
        bcol = bcol_ref[pl.ds(j, 1), :]
        pen = jnp.where(brow == bcol, jnp.float32(0.0), jnp.float32(-1e30))
        for h in range(NHEADS):
            kh = iq_ref[pl.ds(cbase, BC), 2 * D + h * HD: 2 * D + (h + 1) * HD]
            vh = iq_ref[pl.ds(cbase, BC), 3 * D + h * HD: 3 * D + (h + 1) * HD]
            s = lax.dot_general(qs[h], kh, (((1,), (1,)), ((), ())),
                                preferred_element_type=jnp.float32) + pen
            p = jnp.exp(s)
            ls[h] = ls[h] + lax.dot_general(
                p, ones_col, (((1,), (0,)), ((), ())),
                preferred_element_type=jnp.float32)
            accs[h] = accs[h] + lax.dot_general(
                p, vh, (((1,), (0,)), ((), ())),
                preferred_element_type=jnp.float32)
        return tuple(ls + accs)

    carry = lax.fori_loop(lo_ref[r], hi_ref[r], col_step, carry0)
    ls = carry[:NHEADS]
    accs = carry[NHEADS:]
    att = jnp.concatenate([accs[h] / ls[h] for h in range(NHEADS)], axis=1)

    o = jnp.dot(att, wo_ref[...], preferred_element_type=jnp.float32) + bo_ref[...]
    hres = iq_ref[pl.ds(rbase, BR), :D] + o
    mu = jnp.mean(hres, axis=1, keepdims=True)
    dlt = hres - mu
    var = jnp.mean(dlt * dlt, axis=1, keepdims=True)
    o_ref[...] = dlt * lax.rsqrt(var + 1e-5) * lng_ref[...] + lnb_ref[...]


def _attn(lo_blk, hi_blk, incqkv_pad, brow2d, bcol2d, w_o, bo2d, lng2d, lnb2d):
    return pl.pallas_call(
        _attn_body,
        grid=(NRB,),
        in_specs=[
            pl.BlockSpec(memory_space=pltpu.SMEM),
            pl.BlockSpec(memory_space=pltpu.SMEM),
            pl.BlockSpec((N, 4 * D), lambda r: (0, 0)),
            pl.BlockSpec((BR, 1), lambda r: (r, 0)),
            pl.BlockSpec((NCB, BC), lambda r: (0, 0)),
            pl.BlockSpec((D, D), lambda r: (0, 0)),
            pl.BlockSpec((1, D), lambda r: (0, 0)),
            pl.BlockSpec((1, D), lambda r: (0, 0)),
            pl.BlockSpec((1, D), lambda r: (0, 0)),
        ],
        out_specs=pl.BlockSpec((BR, D), lambda r: (r, 0)),
        out_shape=jax.ShapeDtypeStruct((N, D), jnp.float32),
    )(lo_blk, hi_blk, incqkv_pad, brow2d, bcol2d, w_o, bo2d, lng2d, lnb2d)


def kernel(x, edge_index, edge_attr, edge_index2, edge_attr2, batch, num_graphs,
           W_ln, W_c1, W_c2, W_qkv, b_qkv, W_o, b_o, ln_g, ln_b):
    ep = E_PAD - E_SET
    src_all = jnp.concatenate([jnp.pad(edge_index[0], (0, ep)),
                               jnp.pad(edge_index2[0], (0, ep))])
    dst_all = jnp.concatenate([jnp.pad(edge_index[1], (0, ep)),
                               jnp.pad(edge_index2[1], (0, ep))])
    ea_all = jnp.concatenate([jnp.pad(edge_attr, (0, ep)),
                              jnp.pad(edge_attr2, (0, ep))])

    y_both = _sc_scatter_build()(x, src_all, dst_all, ea_all)

    wcat = jnp.concatenate([W_ln, W_c1, W_c2], axis=0)
    w_big = _wfold(wcat, W_qkv)
    bias = jnp.concatenate([jnp.zeros((D,), jnp.float32), b_qkv])[None, :]

    incqkv = _mm(x, y_both, w_big, bias)

    edges = jnp.searchsorted(batch, jnp.arange(NG + 1, dtype=batch.dtype),
                             side="left").astype(jnp.int32)
    starts, ends = edges[:NG], edges[1:]
    rb = batch.reshape(NRB, BR)
    bfirst = rb[:, 0]
    blast = rb[:, -1]
    col_lo = jnp.take(starts, bfirst)
    col_hi = jnp.take(ends, blast)
    lo_blk = (col_lo // BC).astype(jnp.int32)
    hi_blk = ((col_hi + BC - 1) // BC).astype(jnp.int32)

    brow2d = batch[:, None].astype(jnp.int32)
    bcol2d = batch.astype(jnp.int32).reshape(NCB, BC)

    out = _attn(lo_blk, hi_blk, incqkv, brow2d, bcol2d,
                W_o, b_o[None, :], ln_g[None, :], ln_b[None, :])
    return out


# speedup vs baseline: 25.9873x; 1.1056x over previous
import functools

import jax
import jax.numpy as jnp
from jax import lax
from jax.experimental import pallas as pl
from jax.experimental.pallas import tpu as pltpu
from jax.experimental.pallas import tpu_sc as plsc

N = 10000
D = 128
NHEADS = 4
HD = D // NHEADS
NG = 8

SC_CORES = 2
SC_SUBCORES = 16
LANES = 16

E_SET = 320000
CHUNK = 112
N_CHUNKS = 180
E_PER_SUB = N_CHUNKS * CHUNK
E_PAD = E_PER_SUB * SC_SUBCORES
CHUNKS_PER_SET = E_PAD // CHUNK
N_FULL_OUT = N // CHUNK
N_TAIL_OUT = N - N_FULL_OUT * CHUNK
N_OUT_TURNS = N_FULL_OUT // SC_SUBCORES + 1

BR = 400
BC = 400
NRB = N // BR
NCB = N // BC
MM_BR = 2000


def _sc_scatter_build():
    mesh = plsc.VectorSubcoreMesh(core_axis_name="c", subcore_axis_name="s",
                                  num_cores=SC_CORES, num_subcores=SC_SUBCORES)

    @functools.partial(
        pl.kernel,
        mesh=mesh,
        out_type=jax.ShapeDtypeStruct((SC_CORES, N, D), jnp.float32),
        scratch_types=[
            pltpu.VMEM((CHUNK, D), jnp.float32),
            pltpu.VMEM((CHUNK, D), jnp.float32),
            pltpu.VMEM((CHUNK, D), jnp.float32),
            pltpu.VMEM((3, CHUNK), jnp.int32),
            pltpu.VMEM((3, CHUNK), jnp.float32),
            pltpu.VMEM((6, CHUNK), jnp.int32),
            pltpu.VMEM_SHARED((N, D), jnp.float32),
        ] + [pltpu.SemaphoreType.DMA] * 18,
    )
    def sc_kernel(x_hbm, src_hbm, dst_hbm, ea_hbm, out_hbm,
                  r0, r1, r2, src_r, ea_r, dst_r, acc_sh, *sems):
        cc = lax.axis_index("c")
        ss = lax.axis_index("s")
        rows = [r0, r1, r2]
        gsem = list(sems[0:3])
        ssem = list(sems[3:6])
        msem = list(sems[6:9])
        esem = list(sems[9:12])
        dsem = list(sems[12:18])
        rows_v = r0

        def zero_body(i, carry):
            for d in range(D // LANES):
                rows_v[i, pl.ds(d * LANES, LANES)] = jnp.zeros((LANES,), jnp.float32)
            return carry
        lax.fori_loop(0, CHUNK, zero_body, 0)
        for t in range(N_OUT_TURNS):
            idx = ss + SC_SUBCORES * t

            @pl.when(idx < N_FULL_OUT)
            def _():
                pltpu.sync_copy(rows_v, acc_sh.at[pl.ds(idx * CHUNK, CHUNK)])

            @pl.when(idx == N_FULL_OUT)
            def _():
                pltpu.sync_copy(rows_v.at[pl.ds(0, N_TAIL_OUT)],
                                acc_sh.at[pl.ds(N_FULL_OUT * CHUNK, N_TAIL_OUT)])
        plsc.subcore_barrier()

        ebase = (cc * SC_SUBCORES + ss) * E_PER_SUB

        def start_idx(c, j3, j6):
            off = ebase + c * CHUNK
            pltpu.async_copy(src_hbm.at[pl.ds(off, CHUNK)], src_r.at[j3], msem[j3])
            pltpu.async_copy(ea_hbm.at[pl.ds(off, CHUNK)], ea_r.at[j3], esem[j3])
            pltpu.async_copy(dst_hbm.at[pl.ds(off, CHUNK)], dst_r.at[j6], dsem[j6])

        def wait_src(j3):
            pltpu.make_async_copy(src_hbm.at[pl.ds(0, CHUNK)], src_r.at[j3],
                                  msem[j3]).wait()

        def wait_ea(j3):
            pltpu.make_async_copy(ea_hbm.at[pl.ds(0, CHUNK)], ea_r.at[j3],
                                  esem[j3]).wait()

        def wait_dst(j6):
            pltpu.make_async_copy(dst_hbm.at[pl.ds(0, CHUNK)], dst_r.at[j6],
                                  dsem[j6]).wait()

        def start_gather(j3):
            pltpu.async_copy(x_hbm.at[src_r.at[j3]], rows[j3], gsem[j3])

        def wait_gather(j3):
            pltpu.make_async_copy(x_hbm.at[pl.ds(0, CHUNK)], rows[j3],
                                  gsem[j3]).wait()

        def start_scatter(j6, j3):
            pltpu.async_copy(rows[j3], acc_sh.at[dst_r.at[j6]], ssem[j3],
                             add=True)

        def wait_scatter(j3):
            pltpu.make_async_copy(x_hbm.at[pl.ds(0, CHUNK)], rows[j3],
                                  ssem[j3]).wait()

        def scale_chunk(j3):
            buf = rows[j3]

            def scale_body(g, c2):
                ev = ea_r[j3, pl.ds(g * LANES, LANES)]
                for i in range(LANES):
                    eav = ev[i] * jnp.ones((LANES,), jnp.float32)
                    e = g * LANES + i
                    for d in range(D // LANES):
                        sl = pl.ds(d * LANES, LANES)
                        buf[e, sl] = buf[e, sl] * eav
                return c2
            lax.fori_loop(0, CHUNK // LANES, scale_body, 0)

        start_idx(0, 0, 0)
        start_idx(1, 1, 1)
        start_idx(2, 2, 2)
        wait_src(0)
        start_gather(0)
        wait_src(1)
        start_gather(1)
        n_outer = N_CHUNKS // 6

        def outer_body(t, carry):
            for j in range(6):
                c = 6 * t + j
                j3 = j % 3
                jp = (j + 2) % 3
                wait_gather(j3)
                wait_ea(j3)
                scale_chunk(j3)
                wait_dst(j)
                start_scatter(j, j3)

                def prefetch():
                    wait_src(jp)
                    wait_scatter(jp)
                    start_gather(jp)

                def refill():
                    start_idx(c + 3, j3, (j + 3) % 6)

                if j == 0:
                    wait_src(jp)

                    @pl.when(t > 0)
                    def _():
                        wait_scatter(jp)
                    start_gather(jp)
                    refill()
                elif j <= 2:
                    prefetch()
                    refill()
                elif j == 3:
                    prefetch()

                    @pl.when(t < n_outer - 1)
                    def _():
                        refill()
                else:
                    @pl.when(t < n_outer - 1)
                    def _():
                        prefetch()
                        refill()
            return carry
        lax.fori_loop(0, n_outer, outer_body, 0)
        for j3 in range(3):
            wait_scatter(j3)
        plsc.subcore_barrier()

        for t in range(N_OUT_TURNS):
            idx = ss + SC_SUBCORES * t

            @pl.when(idx < N_FULL_OUT)
            def _():
                off = idx * CHUNK
                pltpu.sync_copy(acc_sh.at[pl.ds(off, CHUNK)], rows_v)
                pltpu.sync_copy(rows_v, out_hbm.at[cc, pl.ds(off, CHUNK)])

            @pl.when(idx == N_FULL_OUT)
            def _():
                off = N_FULL_OUT * CHUNK
                pltpu.sync_copy(acc_sh.at[pl.ds(off, N_TAIL_OUT)],
                                rows_v.at[pl.ds(0, N_TAIL_OUT)])
                pltpu.sync_copy(rows_v.at[pl.ds(0, N_TAIL_OUT)],
                                out_hbm.at[cc, pl.ds(off, N_TAIL_OUT)])

    return sc_kernel


def _wfold_body(wcat_ref, wqkv_ref, o_ref):
    o_ref[:, :D] = wcat_ref[...]
    o_ref[:, D:] = jnp.dot(wcat_ref[...], wqkv_ref[...],
                           preferred_element_type=jnp.float32)


def _wfold(wcat, wqkv):
    return pl.pallas_call(
        _wfold_body,
        out_shape=jax.ShapeDtypeStruct((3 * D, 4 * D), jnp.float32),
    )(wcat, wqkv)


def _mm_body(x_ref, y_ref, w_ref, b_ref, o_ref):
    acc = jnp.dot(x_ref[...], w_ref[:D, :], preferred_element_type=jnp.float32)
    acc += jnp.dot(y_ref[0], w_ref[D:2 * D, :], preferred_element_type=jnp.float32)
    acc += jnp.dot(y_ref[1], w_ref[2 * D:, :], preferred_element_type=jnp.float32)
    o_ref[...] = acc + b_ref[...]


def _mm(x, y_both, w_big, bias):
    return pl.pallas_call(
        _mm_body,
        grid=(N // MM_BR,),
        in_specs=[
            pl.BlockSpec((MM_BR, D), lambda i: (i, 0)),
            pl.BlockSpec((2, MM_BR, D), lambda i: (0, i, 0)),
            pl.BlockSpec((3 * D, 4 * D), lambda i: (0, 0)),
            pl.BlockSpec((1, 4 * D), lambda i: (0, 0)),
        ],
        out_specs=pl.BlockSpec((MM_BR, 4 * D), lambda i: (i, 0)),
        out_shape=jax.ShapeDtypeStruct((N, 4 * D), jnp.float32),
    )(x, y_both, w_big, bias)


def _attn_body(lo_ref, hi_ref, iq_ref, brow_ref, bcol_ref,
               wo_ref, bo_ref, lng_ref, lnb_ref, o_ref):
    r = pl.program_id(0)
    rbase = pl.multiple_of(r * BR, BR)
    brow = brow_ref[...]
    scale = 1.0 / jnp.sqrt(jnp.float32(HD))

    qs = [iq_ref[pl.ds(rbase, BR), D + h * HD: D + (h + 1) * HD] * scale
          for h in range(NHEADS)]
    ones_col = jnp.ones((BC, 8), jnp.float32)

    a0 = jnp.zeros((BR, HD + 8), jnp.float32)
    carry0 = (a0,) * NHEADS

    def col_step(j, carry):
        accs = list(carry)
        cbase = pl.multiple_of(j * BC, BC)
        bcol = bcol_ref[pl.ds(j, 1), :]
        pen = jnp.where(brow == bcol, jnp.float32(0.0), jnp.float32(-1e30))
        for h in range(NHEADS):
            kh = iq_ref[pl.ds(cbase, BC), 2 * D + h * HD: 2 * D + (h + 1) * HD]
            vh = iq_ref[pl.ds(cbase, BC), 3 * D + h * HD: 3 * D + (h + 1) * HD]
            p = jnp.exp(lax.dot_general(qs[h], kh, (((1,), (1,)), ((), ())),
                                        preferred_element_type=jnp.float32) + pen)
            vh1 = jnp.concatenate([vh, ones_col], axis=1)
            accs[h] = accs[h] + lax.dot_general(
                p, vh1, (((1,), (0,)), ((), ())),
                preferred_element_type=jnp.float32)
        return tuple(accs)

    accs = lax.fori_loop(lo_ref[r], hi_ref[r], col_step, carry0)
    att = jnp.concatenate(
        [accs[h][:, :HD] / accs[h][:, HD:HD + 1] for h in range(NHEADS)], axis=1)

    o = jnp.dot(att, wo_ref[...], preferred_element_type=jnp.float32) + bo_ref[...]
    hres = iq_ref[pl.ds(rbase, BR), :D] + o
    mu = jnp.mean(hres, axis=1, keepdims=True)
    dlt = hres - mu
    var = jnp.mean(dlt * dlt, axis=1, keepdims=True)
    o_ref[...] = dlt * lax.rsqrt(var + 1e-5) * lng_ref[...] + lnb_ref[...]


def _attn(lo_blk, hi_blk, incqkv_pad, brow2d, bcol2d, w_o, bo2d, lng2d, lnb2d):
    return pl.pallas_call(
        _attn_body,
        grid=(NRB,),
        in_specs=[
            pl.BlockSpec(memory_space=pltpu.SMEM),
            pl.BlockSpec(memory_space=pltpu.SMEM),
            pl.BlockSpec((N, 4 * D), lambda r: (0, 0)),
            pl.BlockSpec((BR, 1), lambda r: (r, 0)),
            pl.BlockSpec((NCB, BC), lambda r: (0, 0)),
            pl.BlockSpec((D, D), lambda r: (0, 0)),
            pl.BlockSpec((1, D), lambda r: (0, 0)),
            pl.BlockSpec((1, D), lambda r: (0, 0)),
            pl.BlockSpec((1, D), lambda r: (0, 0)),
        ],
        out_specs=pl.BlockSpec((BR, D), lambda r: (r, 0)),
        out_shape=jax.ShapeDtypeStruct((N, D), jnp.float32),
    )(lo_blk, hi_blk, incqkv_pad, brow2d, bcol2d, w_o, bo2d, lng2d, lnb2d)


def kernel(x, edge_index, edge_attr, edge_index2, edge_attr2, batch, num_graphs,
           W_ln, W_c1, W_c2, W_qkv, b_qkv, W_o, b_o, ln_g, ln_b):
    ep = E_PAD - E_SET
    src_all = jnp.concatenate([jnp.pad(edge_index[0], (0, ep)),
                               jnp.pad(edge_index2[0], (0, ep))])
    dst_all = jnp.concatenate([jnp.pad(edge_index[1], (0, ep)),
                               jnp.pad(edge_index2[1], (0, ep))])
    ea_all = jnp.concatenate([jnp.pad(edge_attr, (0, ep)),
                              jnp.pad(edge_attr2, (0, ep))])

    y_both = _sc_scatter_build()(x, src_all, dst_all, ea_all)

    wcat = jnp.concatenate([W_ln, W_c1, W_c2], axis=0)
    w_big = _wfold(wcat, W_qkv)
    bias = jnp.concatenate([jnp.zeros((D,), jnp.float32), b_qkv])[None, :]

    incqkv = _mm(x, y_both, w_big, bias)

    edges = jnp.searchsorted(batch, jnp.arange(NG + 1, dtype=batch.dtype),
                             side="left").astype(jnp.int32)
    starts, ends = edges[:NG], edges[1:]
    rb = batch.reshape(NRB, BR)
    bfirst = rb[:, 0]
    blast = rb[:, -1]
    col_lo = jnp.take(starts, bfirst)
    col_hi = jnp.take(ends, blast)
    lo_blk = (col_lo // BC).astype(jnp.int32)
    hi_blk = ((col_hi + BC - 1) // BC).astype(jnp.int32)

    brow2d = batch[:, None].astype(jnp.int32)
    bcol2d = batch.astype(jnp.int32).reshape(NCB, BC)

    out = _attn(lo_blk, hi_blk, incqkv, brow2d, bcol2d,
                W_o, b_o[None, :], ln_g[None, :], ln_b[None, :])
    return out


# speedup vs baseline: 26.0985x; 1.0043x over previous
import functools

import jax
import jax.numpy as jnp
from jax import lax
from jax.experimental import pallas as pl
from jax.experimental.pallas import tpu as pltpu
from jax.experimental.pallas import tpu_sc as plsc

N = 10000
D = 128
NHEADS = 4
HD = D // NHEADS
NG = 8

SC_CORES = 2
SC_SUBCORES = 16
LANES = 16

E_SET = 320000
CHUNK = 48
N_CHUNKS = 420
NRING = 6
E_PER_SUB = N_CHUNKS * CHUNK
E_PAD = E_PER_SUB * SC_SUBCORES
CHUNKS_PER_SET = E_PAD // CHUNK
N_FULL_OUT = N // CHUNK
N_TAIL_OUT = N - N_FULL_OUT * CHUNK
N_OUT_TURNS = N_FULL_OUT // SC_SUBCORES + 1

BR = 400
BC = 400
NRB = N // BR
NCB = N // BC
MM_BR = 2000


def _sc_scatter_build():
    mesh = plsc.VectorSubcoreMesh(core_axis_name="c", subcore_axis_name="s",
                                  num_cores=SC_CORES, num_subcores=SC_SUBCORES)

    @functools.partial(
        pl.kernel,
        mesh=mesh,
        out_type=jax.ShapeDtypeStruct((SC_CORES, N, D), jnp.float32),
        scratch_types=[pltpu.VMEM((CHUNK, D), jnp.float32)] * NRING + [
            pltpu.VMEM((NRING, CHUNK), jnp.int32),
            pltpu.VMEM((NRING, CHUNK), jnp.float32),
            pltpu.VMEM((NRING, CHUNK), jnp.int32),
            pltpu.VMEM_SHARED((N, D), jnp.float32),
        ] + [pltpu.SemaphoreType.DMA] * (5 * NRING),
    )
    def sc_kernel(x_hbm, src_hbm, dst_hbm, ea_hbm, out_hbm, *refs):
        cc = lax.axis_index("c")
        ss = lax.axis_index("s")
        rows = list(refs[0:NRING])
        src_r, ea_r, dst_r, acc_sh = refs[NRING:NRING + 4]
        sems = refs[NRING + 4:]
        gsem = list(sems[0 * NRING:1 * NRING])
        ssem = list(sems[1 * NRING:2 * NRING])
        msem = list(sems[2 * NRING:3 * NRING])
        esem = list(sems[3 * NRING:4 * NRING])
        dsem = list(sems[4 * NRING:5 * NRING])
        rows_v = rows[0]

        def zero_body(i, carry):
            for d in range(D // LANES):
                rows_v[i, pl.ds(d * LANES, LANES)] = jnp.zeros((LANES,), jnp.float32)
            return carry
        lax.fori_loop(0, CHUNK, zero_body, 0)
        for t in range(N_OUT_TURNS):
            idx = ss + SC_SUBCORES * t

            @pl.when(idx < N_FULL_OUT)
            def _():
                pltpu.sync_copy(rows_v, acc_sh.at[pl.ds(idx * CHUNK, CHUNK)])

            @pl.when(idx == N_FULL_OUT)
            def _():
                pltpu.sync_copy(rows_v.at[pl.ds(0, N_TAIL_OUT)],
                                acc_sh.at[pl.ds(N_FULL_OUT * CHUNK, N_TAIL_OUT)])
        plsc.subcore_barrier()

        ebase = (cc * SC_SUBCORES + ss) * E_PER_SUB

        def start_srcea(c, j):
            off = ebase + c * CHUNK
            pltpu.async_copy(src_hbm.at[pl.ds(off, CHUNK)], src_r.at[j], msem[j])
            pltpu.async_copy(ea_hbm.at[pl.ds(off, CHUNK)], ea_r.at[j], esem[j])

        def start_dst(c, j):
            off = ebase + c * CHUNK
            pltpu.async_copy(dst_hbm.at[pl.ds(off, CHUNK)], dst_r.at[j], dsem[j])

        def wait_src(j):
            pltpu.make_async_copy(src_hbm.at[pl.ds(0, CHUNK)], src_r.at[j],
                                  msem[j]).wait()

        def wait_ea(j):
            pltpu.make_async_copy(ea_hbm.at[pl.ds(0, CHUNK)], ea_r.at[j],
                                  esem[j]).wait()

        def wait_dst(j):
            pltpu.make_async_copy(dst_hbm.at[pl.ds(0, CHUNK)], dst_r.at[j],
                                  dsem[j]).wait()

        def start_gather(j):
            pltpu.async_copy(x_hbm.at[src_r.at[j]], rows[j], gsem[j])

        def wait_gather(j):
            pltpu.make_async_copy(x_hbm.at[pl.ds(0, CHUNK)], rows[j],
                                  gsem[j]).wait()

        def start_scatter(j):
            pltpu.async_copy(rows[j], acc_sh.at[dst_r.at[j]], ssem[j], add=True)

        def wait_scatter(j):
            pltpu.make_async_copy(x_hbm.at[pl.ds(0, CHUNK)], rows[j],
                                  ssem[j]).wait()

        def scale_chunk(j):
            buf = rows[j]

            def scale_body(g, c2):
                ev = ea_r[j, pl.ds(g * LANES, LANES)]
                for i in range(LANES):
                    eav = ev[i] * jnp.ones((LANES,), jnp.float32)
                    e = g * LANES + i
                    for d in range(D // LANES):
                        sl = pl.ds(d * LANES, LANES)
                        buf[e, sl] = buf[e, sl] * eav
                return c2
            lax.fori_loop(0, CHUNK // LANES, scale_body, 0)

        for c0 in range(NRING):
            start_srcea(c0, c0)
        for c0 in range(4):
            start_dst(c0, c0)
        for c0 in range(4):
            wait_src(c0)
            start_gather(c0)
        n_outer = N_CHUNKS // NRING

        def outer_body(t, carry):
            for j in range(NRING):
                c = NRING * t + j
                j4 = (j + 4) % NRING
                wait_gather(j)
                wait_ea(j)
                scale_chunk(j)
                wait_dst(j)
                start_scatter(j)

                def prefetch(drain):
                    wait_src(j4)
                    if drain:
                        wait_scatter(j4)
                    start_dst(c + 4, j4)
                    start_gather(j4)

                if j < 2:
                    @pl.when(t > 0)
                    def _():
                        prefetch(True)

                    @pl.when(t == 0)
                    def _():
                        prefetch(False)
                else:
                    @pl.when(t < n_outer - 1)
                    def _():
                        prefetch(True)

                @pl.when(t < n_outer - 1)
                def _():
                    start_srcea(c + NRING, j)
            return carry
        lax.fori_loop(0, n_outer, outer_body, 0)
        for j in range(NRING):
            wait_scatter(j)
        plsc.subcore_barrier()

        for t in range(N_OUT_TURNS):
            idx = ss + SC_SUBCORES * t

            @pl.when(idx < N_FULL_OUT)
            def _():
                off = idx * CHUNK
                pltpu.sync_copy(acc_sh.at[pl.ds(off, CHUNK)], rows_v)
                pltpu.sync_copy(rows_v, out_hbm.at[cc, pl.ds(off, CHUNK)])

            @pl.when(idx == N_FULL_OUT)
            def _():
                off = N_FULL_OUT * CHUNK
                pltpu.sync_copy(acc_sh.at[pl.ds(off, N_TAIL_OUT)],
                                rows_v.at[pl.ds(0, N_TAIL_OUT)])
                pltpu.sync_copy(rows_v.at[pl.ds(0, N_TAIL_OUT)],
                                out_hbm.at[cc, pl.ds(off, N_TAIL_OUT)])

    return sc_kernel


def _wfold_body(wcat_ref, wqkv_ref, o_ref):
    o_ref[:, :D] = wcat_ref[...]
    o_ref[:, D:] = jnp.dot(wcat_ref[...], wqkv_ref[...],
                           preferred_element_type=jnp.float32)


def _wfold(wcat, wqkv):
    return pl.pallas_call(
        _wfold_body,
        out_shape=jax.ShapeDtypeStruct((3 * D, 4 * D), jnp.float32),
    )(wcat, wqkv)


def _mm_body(x_ref, y_ref, w_ref, b_ref, o_ref):
    acc = jnp.dot(x_ref[...], w_ref[:D, :], preferred_element_type=jnp.float32)
    acc += jnp.dot(y_ref[0], w_ref[D:2 * D, :], preferred_element_type=jnp.float32)
    acc += jnp.dot(y_ref[1], w_ref[2 * D:, :], preferred_element_type=jnp.float32)
    o_ref[...] = acc + b_ref[...]


def _mm(x, y_both, w_big, bias):
    return pl.pallas_call(
        _mm_body,
        grid=(N // MM_BR,),
        in_specs=[
            pl.BlockSpec((MM_BR, D), lambda i: (i, 0)),
            pl.BlockSpec((2, MM_BR, D), lambda i: (0, i, 0)),
            pl.BlockSpec((3 * D, 4 * D), lambda i: (0, 0)),
            pl.BlockSpec((1, 4 * D), lambda i: (0, 0)),
        ],
        out_specs=pl.BlockSpec((MM_BR, 4 * D), lambda i: (i, 0)),
        out_shape=jax.ShapeDtypeStruct((N, 4 * D), jnp.float32),
    )(x, y_both, w_big, bias)


def _attn_body(lo_ref, hi_ref, iq_ref, brow_ref, bcol_ref,
               wo_ref, bo_ref, lng_ref, lnb_ref, o_ref):
    r = pl.program_id(0)
    rbase = pl.multiple_of(r * BR, BR)
    brow = brow_ref[...]
    scale = 1.0 / jnp.sqrt(jnp.float32(HD))

    qs = [iq_ref[pl.ds(rbase, BR), D + h * HD: D + (h + 1) * HD] * scale
          for h in range(NHEADS)]
    ones_col = jnp.ones((BC, 8), jnp.float32)

    a0 = jnp.zeros((BR, HD + 8), jnp.float32)
    carry0 = (a0,) * NHEADS

    def col_step(j, carry):
        accs = list(carry)
        cbase = pl.multiple_of(j * BC, BC)
        bcol = bcol_ref[pl.ds(j, 1), :]
        pen = jnp.where(brow == bcol, jnp.float32(0.0), jnp.float32(-1e30))
        for h in range(NHEADS):
            kh = iq_ref[pl.ds(cbase, BC), 2 * D + h * HD: 2 * D + (h + 1) * HD]
            vh = iq_ref[pl.ds(cbase, BC), 3 * D + h * HD: 3 * D + (h + 1) * HD]
            p = jnp.exp(lax.dot_general(qs[h], kh, (((1,), (1,)), ((), ())),
                                        preferred_element_type=jnp.float32) + pen)
            vh1 = jnp.concatenate([vh, ones_col], axis=1)
            accs[h] = accs[h] + lax.dot_general(
                p, vh1, (((1,), (0,)), ((), ())),
                preferred_element_type=jnp.float32)
        return tuple(accs)

    accs = lax.fori_loop(lo_ref[r], hi_ref[r], col_step, carry0)
    att = jnp.concatenate(
        [accs[h][:, :HD] / accs[h][:, HD:HD + 1] for h in range(NHEADS)], axis=1)

    o = jnp.dot(att, wo_ref[...], preferred_element_type=jnp.float32) + bo_ref[...]
    hres = iq_ref[pl.ds(rbase, BR), :D] + o
    mu = jnp.mean(hres, axis=1, keepdims=True)
    dlt = hres - mu
    var = jnp.mean(dlt * dlt, axis=1, keepdims=True)
    o_ref[...] = dlt * lax.rsqrt(var + 1e-5) * lng_ref[...] + lnb_ref[...]


def _attn(lo_blk, hi_blk, incqkv_pad, brow2d, bcol2d, w_o, bo2d, lng2d, lnb2d):
    return pl.pallas_call(
        _attn_body,
        grid=(NRB,),
        in_specs=[
            pl.BlockSpec(memory_space=pltpu.SMEM),
            pl.BlockSpec(memory_space=pltpu.SMEM),
            pl.BlockSpec((N, 4 * D), lambda r: (0, 0)),
            pl.BlockSpec((BR, 1), lambda r: (r, 0)),
            pl.BlockSpec((NCB, BC), lambda r: (0, 0)),
            pl.BlockSpec((D, D), lambda r: (0, 0)),
            pl.BlockSpec((1, D), lambda r: (0, 0)),
            pl.BlockSpec((1, D), lambda r: (0, 0)),
            pl.BlockSpec((1, D), lambda r: (0, 0)),
        ],
        out_specs=pl.BlockSpec((BR, D), lambda r: (r, 0)),
        out_shape=jax.ShapeDtypeStruct((N, D), jnp.float32),
    )(lo_blk, hi_blk, incqkv_pad, brow2d, bcol2d, w_o, bo2d, lng2d, lnb2d)


def kernel(x, edge_index, edge_attr, edge_index2, edge_attr2, batch, num_graphs,
           W_ln, W_c1, W_c2, W_qkv, b_qkv, W_o, b_o, ln_g, ln_b):
    ep = E_PAD - E_SET
    src_all = jnp.concatenate([jnp.pad(edge_index[0], (0, ep)),
                               jnp.pad(edge_index2[0], (0, ep))])
    dst_all = jnp.concatenate([jnp.pad(edge_index[1], (0, ep)),
                               jnp.pad(edge_index2[1], (0, ep))])
    ea_all = jnp.concatenate([jnp.pad(edge_attr, (0, ep)),
                              jnp.pad(edge_attr2, (0, ep))])

    y_both = _sc_scatter_build()(x, src_all, dst_all, ea_all)

    wcat = jnp.concatenate([W_ln, W_c1, W_c2], axis=0)
    w_big = _wfold(wcat, W_qkv)
    bias = jnp.concatenate([jnp.zeros((D,), jnp.float32), b_qkv])[None, :]

    incqkv = _mm(x, y_both, w_big, bias)

    edges = jnp.searchsorted(batch, jnp.arange(NG + 1, dtype=batch.dtype),
                             side="left").astype(jnp.int32)
    starts, ends = edges[:NG], edges[1:]
    rb = batch.reshape(NRB, BR)
    bfirst = rb[:, 0]
    blast = rb[:, -1]
    col_lo = jnp.take(starts, bfirst)
    col_hi = jnp.take(ends, blast)
    lo_blk = (col_lo // BC).astype(jnp.int32)
    hi_blk = ((col_hi + BC - 1) // BC).astype(jnp.int32)

    brow2d = batch[:, None].astype(jnp.int32)
    bcol2d = batch.astype(jnp.int32).reshape(NCB, BC)

    out = _attn(lo_blk, hi_blk, incqkv, brow2d, bcol2d,
                W_o, b_o[None, :], ln_g[None, :], ln_b[None, :])
    return out


# speedup vs baseline: 37.2210x; 1.4262x over previous
import functools

import jax
import jax.numpy as jnp
from jax import lax
from jax.experimental import pallas as pl
from jax.experimental.pallas import tpu as pltpu
from jax.experimental.pallas import tpu_sc as plsc

N = 10000
D = 128
NHEADS = 4
HD = D // NHEADS
NG = 8

SC_CORES = 2
SC_SUBCORES = 16
LANES = 16

E_SET = 320000
CHUNK = 40
N_CHUNKS = 500
NRING = 5
E_PER_SUB = N_CHUNKS * CHUNK
N_FULL_OUT = N // CHUNK
N_OUT_TURNS = -(-N_FULL_OUT // SC_SUBCORES)

BR = 400
BC = 400
NRB = N // BR
NCB = N // BC
MM_BR = 2000


def _sc_scatter_build():
    mesh = plsc.VectorSubcoreMesh(core_axis_name="c", subcore_axis_name="s",
                                  num_cores=SC_CORES, num_subcores=SC_SUBCORES)

    @functools.partial(
        pl.kernel,
        mesh=mesh,
        out_type=jax.ShapeDtypeStruct((SC_CORES, N, D), jnp.float32),
        scratch_types=[pltpu.VMEM((CHUNK, D), jnp.float32)] * NRING + [
            pltpu.VMEM((NRING, CHUNK), jnp.int32),
            pltpu.VMEM((NRING, 48), jnp.float32),
            pltpu.VMEM((NRING, CHUNK), jnp.int32),
            pltpu.VMEM_SHARED((N, D), jnp.float32),
        ] + [pltpu.SemaphoreType.DMA] * (5 * NRING),
    )
    def sc_kernel(x_hbm, ei_hbm, ea_hbm, out_hbm, *refs):
        cc = lax.axis_index("c")
        ss = lax.axis_index("s")
        rows = list(refs[0:NRING])
        src_r, ea_r, dst_r, acc_sh = refs[NRING:NRING + 4]
        sems = refs[NRING + 4:]
        gsem = list(sems[0 * NRING:1 * NRING])
        ssem = list(sems[1 * NRING:2 * NRING])
        msem = list(sems[2 * NRING:3 * NRING])
        esem = list(sems[3 * NRING:4 * NRING])
        dsem = list(sems[4 * NRING:5 * NRING])
        rows_v = rows[0]

        def zero_body(i, carry):
            for d in range(D // LANES):
                rows_v[i, pl.ds(d * LANES, LANES)] = jnp.zeros((LANES,), jnp.float32)
            return carry
        lax.fori_loop(0, CHUNK, zero_body, 0)
        for t in range(N_OUT_TURNS):
            idx = ss + SC_SUBCORES * t

            @pl.when(idx < N_FULL_OUT)
            def _():
                pltpu.sync_copy(rows_v, acc_sh.at[pl.ds(idx * CHUNK, CHUNK)])
        plsc.subcore_barrier()

        sbase = cc * (2 * E_SET) + ss * E_PER_SUB
        dbase = sbase + E_SET
        abase = cc * E_SET + ss * E_PER_SUB

        def start_srcea(c, j):
            off = c * CHUNK
            pltpu.async_copy(ei_hbm.at[pl.ds(sbase + off, CHUNK)],
                             src_r.at[j], msem[j])
            pltpu.async_copy(ea_hbm.at[pl.ds(abase + off, CHUNK)],
                             ea_r.at[j, pl.ds(0, CHUNK)], esem[j])

        def start_dst(c, j):
            pltpu.async_copy(ei_hbm.at[pl.ds(dbase + c * CHUNK, CHUNK)],
                             dst_r.at[j], dsem[j])

        def wait_src(j):
            pltpu.make_async_copy(ei_hbm.at[pl.ds(0, CHUNK)], src_r.at[j],
                                  msem[j]).wait()

        def wait_ea(j):
            pltpu.make_async_copy(ea_hbm.at[pl.ds(0, CHUNK)],
                                  ea_r.at[j, pl.ds(0, CHUNK)], esem[j]).wait()

        def wait_dst(j):
            pltpu.make_async_copy(ei_hbm.at[pl.ds(0, CHUNK)], dst_r.at[j],
                                  dsem[j]).wait()

        def start_gather(j):
            pltpu.async_copy(x_hbm.at[src_r.at[j]], rows[j], gsem[j])

        def wait_gather(j):
            pltpu.make_async_copy(x_hbm.at[pl.ds(0, CHUNK)], rows[j],
                                  gsem[j]).wait()

        def start_scatter(j):
            pltpu.async_copy(rows[j], acc_sh.at[dst_r.at[j]], ssem[j], add=True)

        def wait_scatter(j):
            pltpu.make_async_copy(x_hbm.at[pl.ds(0, CHUNK)], rows[j],
                                  ssem[j]).wait()

        def scale_chunk(j):
            buf = rows[j]

            def scale_group(g, n_lanes, c2=0):
                ev = ea_r[j, pl.ds(g * LANES, LANES)]
                for i in range(n_lanes):
                    eav = ev[i] * jnp.ones((LANES,), jnp.float32)
                    e = g * LANES + i
                    for d in range(D // LANES):
                        sl = pl.ds(d * LANES, LANES)
                        buf[e, sl] = buf[e, sl] * eav
                return c2

            lax.fori_loop(0, CHUNK // LANES,
                          lambda g, c2: scale_group(g, LANES, c2), 0)
            if CHUNK % LANES:
                scale_group(CHUNK // LANES, CHUNK % LANES)

        for c0 in range(NRING):
            start_srcea(c0, c0)
        for c0 in range(3):
            start_dst(c0, c0)
        for c0 in range(3):
            wait_src(c0)
            start_gather(c0)
        n_outer = N_CHUNKS // NRING

        def outer_body(t, carry):
            for j in range(NRING):
                c = NRING * t + j
                j3 = (j + 3) % NRING
                wait_gather(j)
                wait_ea(j)
                scale_chunk(j)
                wait_dst(j)
                start_scatter(j)

                def prefetch(drain):
                    wait_src(j3)
                    if drain:
                        wait_scatter(j3)
                    start_dst(c + 3, j3)
                    start_gather(j3)

                if j < 2:
                    @pl.when(t > 0)
                    def _():
                        prefetch(True)

                    @pl.when(t == 0)
                    def _():
                        prefetch(False)
                else:
                    @pl.when(t < n_outer - 1)
                    def _():
                        prefetch(True)

                @pl.when(t < n_outer - 1)
                def _():
                    start_srcea(c + NRING, j)
            return carry
        lax.fori_loop(0, n_outer, outer_body, 0)
        for j in range(NRING):
            wait_scatter(j)
        plsc.subcore_barrier()

        for t in range(N_OUT_TURNS):
            idx = ss + SC_SUBCORES * t

            @pl.when(idx < N_FULL_OUT)
            def _():
                off = idx * CHUNK
                pltpu.sync_copy(acc_sh.at[pl.ds(off, CHUNK)], rows_v)
                pltpu.sync_copy(rows_v, out_hbm.at[cc, pl.ds(off, CHUNK)])

    return sc_kernel


def _wfold_body(wcat_ref, wqkv_ref, o_ref):
    o_ref[:, :D] = wcat_ref[...]
    o_ref[:, D:] = jnp.dot(wcat_ref[...], wqkv_ref[...],
                           preferred_element_type=jnp.float32)


def _wfold(wcat, wqkv):
    return pl.pallas_call(
        _wfold_body,
        out_shape=jax.ShapeDtypeStruct((3 * D, 4 * D), jnp.float32),
    )(wcat, wqkv)


def _mm_body(x_ref, y_ref, w_ref, b_ref, o_ref):
    acc = jnp.dot(x_ref[...], w_ref[:D, :], preferred_element_type=jnp.float32)
    acc += jnp.dot(y_ref[0], w_ref[D:2 * D, :], preferred_element_type=jnp.float32)
    acc += jnp.dot(y_ref[1], w_ref[2 * D:, :], preferred_element_type=jnp.float32)
    o_ref[...] = acc + b_ref[...]


def _mm(x, y_both, w_big, bias):
    return pl.pallas_call(
        _mm_body,
        grid=(N // MM_BR,),
        in_specs=[
            pl.BlockSpec((MM_BR, D), lambda i: (i, 0)),
            pl.BlockSpec((2, MM_BR, D), lambda i: (0, i, 0)),
            pl.BlockSpec((3 * D, 4 * D), lambda i: (0, 0)),
            pl.BlockSpec((1, 4 * D), lambda i: (0, 0)),
        ],
        out_specs=pl.BlockSpec((MM_BR, 4 * D), lambda i: (i, 0)),
        out_shape=jax.ShapeDtypeStruct((N, 4 * D), jnp.float32),
    )(x, y_both, w_big, bias)


def _attn_body(lo_ref, hi_ref, iq_ref, brow_ref, bcol_ref,
               wo_ref, bo_ref, lng_ref, lnb_ref, o_ref):
    r = pl.program_id(0)
    rbase = pl.multiple_of(r * BR, BR)
    brow = brow_ref[...]
    scale = 1.0 / jnp.sqrt(jnp.float32(HD))

    qs = [iq_ref[pl.ds(rbase, BR), D + h * HD: D + (h + 1) * HD] * scale
          for h in range(NHEADS)]
    ones_col = jnp.ones((BC, 8), jnp.float32)

    a0 = jnp.zeros((BR, HD + 8), jnp.float32)
    carry0 = (a0,) * NHEADS

    def col_step(j, carry):
        accs = list(carry)
        cbase = pl.multiple_of(j * BC, BC)
        bcol = bcol_ref[pl.ds(j, 1), :]
        pen = jnp.where(brow == bcol, jnp.float32(0.0), jnp.float32(-1e30))
        for h in range(NHEADS):
            kh = iq_ref[pl.ds(cbase, BC), 2 * D + h * HD: 2 * D + (h + 1) * HD]
            vh = iq_ref[pl.ds(cbase, BC), 3 * D + h * HD: 3 * D + (h + 1) * HD]
            p = jnp.exp(lax.dot_general(qs[h], kh, (((1,), (1,)), ((), ())),
                                        preferred_element_type=jnp.float32) + pen)
            vh1 = jnp.concatenate([vh, ones_col], axis=1)
            accs[h] = accs[h] + lax.dot_general(
                p, vh1, (((1,), (0,)), ((), ())),
                preferred_element_type=jnp.float32)
        return tuple(accs)

    accs = lax.fori_loop(lo_ref[r], hi_ref[r], col_step, carry0)
    att = jnp.concatenate(
        [accs[h][:, :HD] / accs[h][:, HD:HD + 1] for h in range(NHEADS)], axis=1)

    o = jnp.dot(att, wo_ref[...], preferred_element_type=jnp.float32) + bo_ref[...]
    hres = iq_ref[pl.ds(rbase, BR), :D] + o
    mu = jnp.mean(hres, axis=1, keepdims=True)
    dlt = hres - mu
    var = jnp.mean(dlt * dlt, axis=1, keepdims=True)
    o_ref[...] = dlt * lax.rsqrt(var + 1e-5) * lng_ref[...] + lnb_ref[...]


def _attn(lo_blk, hi_blk, incqkv_pad, brow2d, bcol2d, w_o, bo2d, lng2d, lnb2d):
    return pl.pallas_call(
        _attn_body,
        grid=(NRB,),
        in_specs=[
            pl.BlockSpec(memory_space=pltpu.SMEM),
            pl.BlockSpec(memory_space=pltpu.SMEM),
            pl.BlockSpec((N, 4 * D), lambda r: (0, 0)),
            pl.BlockSpec((BR, 1), lambda r: (r, 0)),
            pl.BlockSpec((NCB, BC), lambda r: (0, 0)),
            pl.BlockSpec((D, D), lambda r: (0, 0)),
            pl.BlockSpec((1, D), lambda r: (0, 0)),
            pl.BlockSpec((1, D), lambda r: (0, 0)),
            pl.BlockSpec((1, D), lambda r: (0, 0)),
        ],
        out_specs=pl.BlockSpec((BR, D), lambda r: (r, 0)),
        out_shape=jax.ShapeDtypeStruct((N, D), jnp.float32),
    )(lo_blk, hi_blk, incqkv_pad, brow2d, bcol2d, w_o, bo2d, lng2d, lnb2d)


def kernel(x, edge_index, edge_attr, edge_index2, edge_attr2, batch, num_graphs,
           W_ln, W_c1, W_c2, W_qkv, b_qkv, W_o, b_o, ln_g, ln_b):
    ei_all = jnp.concatenate([edge_index.reshape(-1), edge_index2.reshape(-1)])
    ea_all = jnp.concatenate([edge_attr, edge_attr2])

    y_both = _sc_scatter_build()(x, ei_all, ea_all)

    wcat = jnp.concatenate([W_ln, W_c1, W_c2], axis=0)
    w_big = _wfold(wcat, W_qkv)
    bias = jnp.concatenate([jnp.zeros((D,), jnp.float32), b_qkv])[None, :]

    incqkv = _mm(x, y_both, w_big, bias)

    edges = jnp.searchsorted(batch, jnp.arange(NG + 1, dtype=batch.dtype),
                             side="left").astype(jnp.int32)
    starts, ends = edges[:NG], edges[1:]
    rb = batch.reshape(NRB, BR)
    bfirst = rb[:, 0]
    blast = rb[:, -1]
    col_lo = jnp.take(starts, bfirst)
    col_hi = jnp.take(ends, blast)
    lo_blk = (col_lo // BC).astype(jnp.int32)
    hi_blk = ((col_hi + BC - 1) // BC).astype(jnp.int32)

    brow2d = batch[:, None].astype(jnp.int32)
    bcol2d = batch.astype(jnp.int32).reshape(NCB, BC)

    out = _attn(lo_blk, hi_blk, incqkv, brow2d, bcol2d,
                W_o, b_o[None, :], ln_g[None, :], ln_b[None, :])
    return out
